# Initial kernel scaffold; baseline (speedup 1.0000x reference)
import jax
import jax.numpy as jnp
from jax.experimental import pallas as pl


def kernel(x, edge_index, W1, b1, W2, b2, W3, b3):
    raise NotImplementedError("write your pallas kernel here")


# speedup vs baseline: 12.4884x; 12.4884x over previous
import functools

import jax
import jax.numpy as jnp
from jax import lax
from jax.experimental import pallas as pl
from jax.experimental.pallas import tpu as pltpu
from jax.experimental.pallas import tpu_sc as plsc

_N = 10000
_E = 160000
_D = 256
_H = 128
_NCHUNK = _E // _H
_NTILE = 16
_RPT = _N // _NTILE
_NHALF = _N // 2
_DEGPAD = 5120
_BM = 1000

_mesh = plsc.VectorSubcoreMesh(core_axis_name="c", subcore_axis_name="s")


def _zero16():
    return jnp.zeros((16,), jnp.float32)


def _one16():
    return jnp.ones((16,), jnp.float32)


def _sc_deg_body(dst_hbm, deg_hbm, idxb, lib, onesb, stage, shared_deg):
    c = lax.axis_index("c")
    s = lax.axis_index("s")

    for k in range(8):
        onesb[pl.ds(k * 16, 16)] = _one16()
    for k in range(20):
        stage[pl.ds(k * 16, 16)] = _zero16()
    pltpu.sync_copy(stage, shared_deg.at[pl.ds(s * 320, 320)])
    plsc.subcore_barrier()

    node_base = c * _NHALF

    @pl.loop(0, 79)
    def _chunks(j):
        chunk = s + j * _NTILE

        @pl.when(chunk < _NCHUNK)
        def _():
            pltpu.sync_copy(dst_hbm.at[chunk], idxb)
            for k in range(8):
                v = idxb[pl.ds(k * 16, 16)] - node_base
                ok = (v >= 0) & (v < _NHALF)
                lib[pl.ds(k * 16, 16)] = jnp.where(ok, v, _NHALF)
            pltpu.sync_copy(onesb, shared_deg.at[lib], add=True)

    plsc.subcore_barrier()
    pltpu.sync_copy(shared_deg.at[pl.ds(s * 320, 320)], stage)
    pltpu.sync_copy(stage, deg_hbm.at[pl.ds(c * _DEGPAD + s * 320, 320)])


_sc_deg = functools.partial(
    pl.kernel,
    out_type=jax.ShapeDtypeStruct((2 * _DEGPAD,), jnp.float32),
    mesh=_mesh,
    scratch_types=[
        pltpu.VMEM((_H,), jnp.int32),
        pltpu.VMEM((_H,), jnp.int32),
        pltpu.VMEM((_H,), jnp.float32),
        pltpu.VMEM((320,), jnp.float32),
        pltpu.VMEM_SHARED((_DEGPAD + 16,), jnp.float32),
    ],
)(_sc_deg_body)


def _sc_scatter_body(hp_hbm, src_hbm, dst_hbm, acc_hbm,
                     srcb0, srcb1, dstb0, dstb1, rows0, rows1, zbuf,
                     shared_acc, sem0, sem1):
    c = lax.axis_index("c")
    s = lax.axis_index("s")
    row_off = c * _N

    srcbufs = (srcb0, srcb1)
    dstbufs = (dstb0, dstb1)
    rowbufs = (rows0, rows1)
    sems = (sem0, sem1)

    @pl.loop(0, _H)
    def _z(r):
        for k in range(8):
            zbuf[r, pl.ds(k * 16, 16)] = _zero16()

    base = s * 640

    def _zero_at(off, rows, buf):
        pltpu.sync_copy(buf.at[pl.ds(0, rows), :],
                        shared_acc.at[pl.ds(off, rows), :])

    for k in range(5):
        @pl.when((s < 15) | (k < 3))
        def _():
            _zero_at(base + k * _H, _H, zbuf)

    @pl.when(s == 15)
    def _():
        _zero_at(base + 3 * _H, 16, zbuf)

    plsc.subcore_barrier()

    nj = ((_NCHUNK - 1 - s) // _NTILE) + 1

    def _load_and_fire(j, p):
        chunk = s + j * _NTILE
        pltpu.sync_copy(src_hbm.at[chunk], srcbufs[p])
        pltpu.sync_copy(dst_hbm.at[chunk], dstbufs[p])
        for k in range(8):
            srcbufs[p][pl.ds(k * 16, 16)] = (
                srcbufs[p][pl.ds(k * 16, 16)] + row_off)
        pltpu.async_copy(hp_hbm.at[srcbufs[p]], rowbufs[p], sems[p])

    _load_and_fire(0, 0)

    @pl.loop(0, 40)
    def _pairs(jp):
        for b in range(2):
            j = jp * 2 + b
            p = b

            @pl.when(j < nj)
            def _():
                @pl.when(j + 1 < nj)
                def _():
                    _load_and_fire(j + 1, 1 - p)

                pltpu.make_async_copy(
                    hp_hbm.at[srcbufs[p]], rowbufs[p], sems[p]).wait()
                pltpu.sync_copy(rowbufs[p], shared_acc.at[dstbufs[p]],
                                add=True)

    plsc.subcore_barrier()

    def _wb(off, rows):
        pltpu.sync_copy(shared_acc.at[pl.ds(off, rows), :],
                        zbuf.at[pl.ds(0, rows), :])
        pltpu.sync_copy(zbuf.at[pl.ds(0, rows), :],
                        acc_hbm.at[pl.ds(row_off + off, rows), :])

    for k in range(5):
        @pl.when((s < 15) | (k < 3))
        def _():
            _wb(base + k * _H, _H)

    @pl.when(s == 15)
    def _():
        _wb(base + 3 * _H, 16)


_sc_scatter = functools.partial(
    pl.kernel,
    out_type=jax.ShapeDtypeStruct((2 * _N, _H), jnp.float32),
    mesh=_mesh,
    scratch_types=[
        pltpu.VMEM((_H,), jnp.int32),
        pltpu.VMEM((_H,), jnp.int32),
        pltpu.VMEM((_H,), jnp.int32),
        pltpu.VMEM((_H,), jnp.int32),
        pltpu.VMEM((_H, _H), jnp.float32),
        pltpu.VMEM((_H, _H), jnp.float32),
        pltpu.VMEM((_H, _H), jnp.float32),
        pltpu.VMEM_SHARED((_N, _H), jnp.float32),
        pltpu.SemaphoreType.DMA,
        pltpu.SemaphoreType.DMA,
    ],
)(_sc_scatter_body)


def _dis(deg_blk):
    return lax.rsqrt(deg_blk + 1.0)


def _tc1_body(x_ref, w_ref, deg_ref, hp_ref):
    dis = _dis(deg_ref[...])
    h = jnp.dot(x_ref[...], w_ref[...], preferred_element_type=jnp.float32)
    hp = h * dis
    hp_ref[0] = hp[:, :_H]
    hp_ref[1] = hp[:, _H:]


def _tc_mid_body(acc_ref, hp_ref, deg_ref, b_ref, w_ref, out_ref):
    dis = _dis(deg_ref[...])
    b = b_ref[...]
    hin_a = jnp.maximum(dis * (acc_ref[0] + hp_ref[0]) + b[:, :_H], 0.0)
    hin_b = jnp.maximum(dis * (acc_ref[1] + hp_ref[1]) + b[:, _H:], 0.0)
    hin = jnp.concatenate([hin_a, hin_b], axis=1)
    h = jnp.dot(hin, w_ref[...], preferred_element_type=jnp.float32)
    hp = h * dis
    out_ref[0] = hp[:, :_H]
    out_ref[1] = hp[:, _H:]


def _tc_final_body(acc_ref, hp_ref, deg_ref, b_ref, out_ref):
    dis = _dis(deg_ref[...])
    out_a = dis * (acc_ref[0] + hp_ref[0]) + b_ref[...][:, :_H]
    out_b = dis * (acc_ref[1] + hp_ref[1]) + b_ref[...][:, _H:]
    out_ref[...] = jnp.concatenate([out_a, out_b], axis=1)


_GRID = _N // _BM

_spec_rows = pl.BlockSpec((_BM, _D), lambda i: (i, 0))
_spec_halves = pl.BlockSpec((2, _BM, _H), lambda i: (0, i, 0))
_spec_deg = pl.BlockSpec((_BM, 1), lambda i: (i, 0))
_spec_w = pl.BlockSpec((_D, _D), lambda i: (0, 0))
_spec_b = pl.BlockSpec((1, _D), lambda i: (0, 0))

_tc1 = pl.pallas_call(
    _tc1_body,
    grid=(_GRID,),
    in_specs=[_spec_rows, _spec_w, _spec_deg],
    out_specs=_spec_halves,
    out_shape=jax.ShapeDtypeStruct((2, _N, _H), jnp.float32),
)

_tc_mid = pl.pallas_call(
    _tc_mid_body,
    grid=(_GRID,),
    in_specs=[_spec_halves, _spec_halves, _spec_deg, _spec_b, _spec_w],
    out_specs=_spec_halves,
    out_shape=jax.ShapeDtypeStruct((2, _N, _H), jnp.float32),
)

_tc_final = pl.pallas_call(
    _tc_final_body,
    grid=(_GRID,),
    in_specs=[_spec_halves, _spec_halves, _spec_deg, _spec_b],
    out_specs=_spec_rows,
    out_shape=jax.ShapeDtypeStruct((_N, _D), jnp.float32),
)


def kernel(x, edge_index, W1, b1, W2, b2, W3, b3):
    src2d = edge_index[0].reshape(_NCHUNK, _H)
    dst2d = edge_index[1].reshape(_NCHUNK, _H)

    deg_raw = _sc_deg(dst2d)
    degc = deg_raw.reshape(2, _DEGPAD)[:, :_NHALF].reshape(_N, 1)

    b1r = b1.reshape(1, _D)
    b2r = b2.reshape(1, _D)
    b3r = b3.reshape(1, _D)

    hp1 = _tc1(x, W1, degc)
    acc1 = _sc_scatter(hp1.reshape(2 * _N, _H), src2d, dst2d)
    hp2 = _tc_mid(acc1.reshape(2, _N, _H), hp1, degc, b1r, W2)
    acc2 = _sc_scatter(hp2.reshape(2 * _N, _H), src2d, dst2d)
    hp3 = _tc_mid(acc2.reshape(2, _N, _H), hp2, degc, b2r, W3)
    acc3 = _sc_scatter(hp3.reshape(2 * _N, _H), src2d, dst2d)
    out = _tc_final(acc3.reshape(2, _N, _H), hp3, degc, b3r)
    return out


# speedup vs baseline: 15.8021x; 1.2653x over previous
import functools

import jax
import jax.numpy as jnp
from jax import lax
from jax.experimental import pallas as pl
from jax.experimental.pallas import tpu as pltpu
from jax.experimental.pallas import tpu_sc as plsc

_N = 10000
_E = 160000
_D = 256
_H = 128
_NCHUNK = _E // _H
_NTILE = 16
_RPT = _N // _NTILE
_NHALF = _N // 2
_DEGPAD = 5120
_BM = 1000

_mesh = plsc.VectorSubcoreMesh(core_axis_name="c", subcore_axis_name="s")


def _zero16():
    return jnp.zeros((16,), jnp.float32)


def _one16():
    return jnp.ones((16,), jnp.float32)


def _sc_deg_body(dstf_hbm, deg_hbm, blk0, blk1, lib, onesb, stage,
                 tailb, shared_deg, bsem0, bsem1):
    c = lax.axis_index("c")
    s = lax.axis_index("s")
    blks = (blk0, blk1)
    bsems = (bsem0, bsem1)

    for k in range(8):
        onesb[pl.ds(k * 16, 16)] = _one16()
    for k in range(20):
        stage[pl.ds(k * 16, 16)] = _zero16()
    pltpu.sync_copy(stage, shared_deg.at[pl.ds(s * 320, 320)])
    plsc.subcore_barrier()

    node_base = c * _NHALF
    nblk = _NCHUNK // 8

    def _scatter_ones(src_ref, r):
        for k in range(8):
            v = src_ref[pl.ds(r * _H + k * 16, 16)] - node_base
            ok = (v >= 0) & (v < _NHALF)
            lib[pl.ds(k * 16, 16)] = jnp.where(ok, v, _NHALF)
        pltpu.sync_copy(onesb, shared_deg.at[lib], add=True)

    def _fire(t, p):
        pltpu.async_copy(dstf_hbm.at[pl.ds(t * 8 * _H, 8 * _H)], blks[p],
                         bsems[p])

    _fire(s, 0)

    @pl.loop(0, 5)
    def _groups(g):
        for b in range(2):
            i = g * 2 + b
            t = s + i * _NTILE

            @pl.when(t < nblk)
            def _():
                tn = t + _NTILE

                @pl.when(tn < nblk)
                def _():
                    _fire(tn, 1 - b)

                pltpu.make_async_copy(
                    dstf_hbm.at[pl.ds(t * 8 * _H, 8 * _H)], blks[b],
                    bsems[b]).wait()
                for r in range(8):
                    _scatter_ones(blks[b], r)

    @pl.when(s == 0)
    def _():
        pltpu.sync_copy(dstf_hbm.at[pl.ds(8 * nblk * _H, 2 * _H)], tailb)
        for r in range(2):
            _scatter_ones(tailb, r)

    plsc.subcore_barrier()
    pltpu.sync_copy(shared_deg.at[pl.ds(s * 320, 320)], stage)
    pltpu.sync_copy(stage, deg_hbm.at[pl.ds(c * _DEGPAD + s * 320, 320)])


_sc_deg = functools.partial(
    pl.kernel,
    out_type=jax.ShapeDtypeStruct((2 * _DEGPAD,), jnp.float32),
    mesh=_mesh,
    scratch_types=[
        pltpu.VMEM((8 * _H,), jnp.int32),
        pltpu.VMEM((8 * _H,), jnp.int32),
        pltpu.VMEM((_H,), jnp.int32),
        pltpu.VMEM((_H,), jnp.float32),
        pltpu.VMEM((320,), jnp.float32),
        pltpu.VMEM((2 * _H,), jnp.int32),
        pltpu.VMEM_SHARED((_DEGPAD + 16,), jnp.float32),
        pltpu.SemaphoreType.DMA,
        pltpu.SemaphoreType.DMA,
    ],
)(_sc_deg_body)


def _sc_scatter_body(hp_hbm, src_hbm, dst_hbm, acc_hbm,
                     srcb0, srcb1, srcb2,
                     dstb0, dstb1, dstb2,
                     rows0, rows1, rows2, shared_acc,
                     isem0, isem1, isem2,
                     gsem0, gsem1, gsem2,
                     ssem0, ssem1, ssem2):
    c = lax.axis_index("c")
    s = lax.axis_index("s")
    row_off = c * _N

    srcbufs = (srcb0, srcb1, srcb2)
    dstbufs = (dstb0, dstb1, dstb2)
    rowbufs = (rows0, rows1, rows2)
    isems = (isem0, isem1, isem2)
    gsems = (gsem0, gsem1, gsem2)
    ssems = (ssem0, ssem1, ssem2)
    zbuf = rows0

    @pl.loop(0, _H)
    def _z(r):
        for k in range(8):
            zbuf[r, pl.ds(k * 16, 16)] = _zero16()

    base = s * 640

    def _zero_at(off, rows, buf):
        pltpu.sync_copy(buf.at[pl.ds(0, rows), :],
                        shared_acc.at[pl.ds(off, rows), :])

    for k in range(5):
        @pl.when((s < 15) | (k < 3))
        def _():
            _zero_at(base + k * _H, _H, zbuf)

    @pl.when(s == 15)
    def _():
        _zero_at(base + 3 * _H, 16, zbuf)

    plsc.subcore_barrier()

    nj = ((_NCHUNK - 1 - s) // _NTILE) + 1

    def _chunk(j):
        return s + j * _NTILE

    def _idx_load(j, p):
        pltpu.async_copy(src_hbm.at[_chunk(j)], srcbufs[p], isems[p])
        pltpu.async_copy(dst_hbm.at[_chunk(j)], dstbufs[p], isems[p])

    def _idx_wait(j, p):
        pltpu.make_async_copy(src_hbm.at[_chunk(j)], srcbufs[p],
                              isems[p]).wait()
        pltpu.make_async_copy(dst_hbm.at[_chunk(j)], dstbufs[p],
                              isems[p]).wait()

    def _gather_fire(p):
        for k in range(8):
            srcbufs[p][pl.ds(k * 16, 16)] = (
                srcbufs[p][pl.ds(k * 16, 16)] + row_off)
        pltpu.async_copy(hp_hbm.at[srcbufs[p]], rowbufs[p], gsems[p])

    def _scatter_fire(p):
        pltpu.make_async_copy(hp_hbm.at[srcbufs[p]], rowbufs[p],
                              gsems[p]).wait()
        pltpu.async_copy(rowbufs[p], shared_acc.at[dstbufs[p]], ssems[p],
                         add=True)

    def _scatter_wait(p):
        pltpu.make_async_copy(rowbufs[p], shared_acc.at[dstbufs[p]],
                              ssems[p]).wait()

    @pl.loop(0, 28)
    def _steps(g):
        for p in range(3):
            j = g * 3 + p

            @pl.when((j >= 3) & (j - 3 < nj))
            def _():
                _scatter_wait(p)

            @pl.when(j < nj)
            def _():
                _idx_load(j, p)

            q = (p - 1) % 3

            @pl.when((j >= 1) & (j - 1 < nj))
            def _():
                _idx_wait(j - 1, q)
                _gather_fire(q)

            r = (p - 2) % 3

            @pl.when((j >= 2) & (j - 2 < nj))
            def _():
                _scatter_fire(r)

    plsc.subcore_barrier()

    def _wb(off, rows):
        pltpu.sync_copy(shared_acc.at[pl.ds(off, rows), :],
                        zbuf.at[pl.ds(0, rows), :])
        pltpu.sync_copy(zbuf.at[pl.ds(0, rows), :],
                        acc_hbm.at[pl.ds(row_off + off, rows), :])

    for k in range(5):
        @pl.when((s < 15) | (k < 3))
        def _():
            _wb(base + k * _H, _H)

    @pl.when(s == 15)
    def _():
        _wb(base + 3 * _H, 16)


_sc_scatter = functools.partial(
    pl.kernel,
    out_type=jax.ShapeDtypeStruct((2 * _N, _H), jnp.float32),
    mesh=_mesh,
    scratch_types=(
        [pltpu.VMEM((_H,), jnp.int32)] * 6
        + [pltpu.VMEM((_H, _H), jnp.float32)] * 3
        + [pltpu.VMEM_SHARED((_N, _H), jnp.float32)]
        + [pltpu.SemaphoreType.DMA] * 9
    ),
)(_sc_scatter_body)


def _dis(deg_blk):
    return lax.rsqrt(deg_blk + 1.0)


def _tc1_body(x_ref, w_ref, deg_ref, hp_ref):
    dis = _dis(deg_ref[...])
    h = jnp.dot(x_ref[...], w_ref[...], preferred_element_type=jnp.float32)
    hp = h * dis
    hp_ref[0] = hp[:, :_H]
    hp_ref[1] = hp[:, _H:]


def _tc_mid_body(acc_ref, hp_ref, deg_ref, b_ref, w_ref, out_ref):
    dis = _dis(deg_ref[...])
    b = b_ref[...]
    hin_a = jnp.maximum(dis * (acc_ref[0] + hp_ref[0]) + b[:, :_H], 0.0)
    hin_b = jnp.maximum(dis * (acc_ref[1] + hp_ref[1]) + b[:, _H:], 0.0)
    hin = jnp.concatenate([hin_a, hin_b], axis=1)
    h = jnp.dot(hin, w_ref[...], preferred_element_type=jnp.float32)
    hp = h * dis
    out_ref[0] = hp[:, :_H]
    out_ref[1] = hp[:, _H:]


def _tc_final_body(acc_ref, hp_ref, deg_ref, b_ref, out_ref):
    dis = _dis(deg_ref[...])
    out_a = dis * (acc_ref[0] + hp_ref[0]) + b_ref[...][:, :_H]
    out_b = dis * (acc_ref[1] + hp_ref[1]) + b_ref[...][:, _H:]
    out_ref[...] = jnp.concatenate([out_a, out_b], axis=1)


_GRID = _N // _BM

_spec_rows = pl.BlockSpec((_BM, _D), lambda i: (i, 0))
_spec_halves = pl.BlockSpec((2, _BM, _H), lambda i: (0, i, 0))
_spec_deg = pl.BlockSpec((_BM, 1), lambda i: (i, 0))
_spec_w = pl.BlockSpec((_D, _D), lambda i: (0, 0))
_spec_b = pl.BlockSpec((1, _D), lambda i: (0, 0))

_tc1 = pl.pallas_call(
    _tc1_body,
    grid=(_GRID,),
    in_specs=[_spec_rows, _spec_w, _spec_deg],
    out_specs=_spec_halves,
    out_shape=jax.ShapeDtypeStruct((2, _N, _H), jnp.float32),
)

_tc_mid = pl.pallas_call(
    _tc_mid_body,
    grid=(_GRID,),
    in_specs=[_spec_halves, _spec_halves, _spec_deg, _spec_b, _spec_w],
    out_specs=_spec_halves,
    out_shape=jax.ShapeDtypeStruct((2, _N, _H), jnp.float32),
)

_tc_final = pl.pallas_call(
    _tc_final_body,
    grid=(_GRID,),
    in_specs=[_spec_halves, _spec_halves, _spec_deg, _spec_b],
    out_specs=_spec_rows,
    out_shape=jax.ShapeDtypeStruct((_N, _D), jnp.float32),
)


def kernel(x, edge_index, W1, b1, W2, b2, W3, b3):
    src2d = edge_index[0].reshape(_NCHUNK, _H)
    dst2d = edge_index[1].reshape(_NCHUNK, _H)

    deg_raw = _sc_deg(edge_index[1])
    degc = deg_raw.reshape(2, _DEGPAD)[:, :_NHALF].reshape(_N, 1)

    b1r = b1.reshape(1, _D)
    b2r = b2.reshape(1, _D)
    b3r = b3.reshape(1, _D)

    hp1 = _tc1(x, W1, degc)
    acc1 = _sc_scatter(hp1.reshape(2 * _N, _H), src2d, dst2d)
    hp2 = _tc_mid(acc1.reshape(2, _N, _H), hp1, degc, b1r, W2)
    acc2 = _sc_scatter(hp2.reshape(2 * _N, _H), src2d, dst2d)
    hp3 = _tc_mid(acc2.reshape(2, _N, _H), hp2, degc, b2r, W3)
    acc3 = _sc_scatter(hp3.reshape(2 * _N, _H), src2d, dst2d)
    out = _tc_final(acc3.reshape(2, _N, _H), hp3, degc, b3r)
    return out


# speedup vs baseline: 20.8479x; 1.3193x over previous
import functools

import jax
import jax.numpy as jnp
from jax import lax
from jax.experimental import pallas as pl
from jax.experimental.pallas import tpu as pltpu
from jax.experimental.pallas import tpu_sc as plsc

_N = 10000
_E = 160000
_D = 256
_H = 128
_NCHUNK = _E // _H
_NTILE = 16
_RPT = _N // _NTILE
_NHALF = _N // 2
_DEGPAD = 5120
_CH = 80
_NCH2 = _E // _CH
_BM = 1024

_mesh = plsc.VectorSubcoreMesh(core_axis_name="c", subcore_axis_name="s")


def _zero16():
    return jnp.zeros((16,), jnp.float32)


def _one16():
    return jnp.ones((16,), jnp.float32)


def _sc_deg_body(dstf_hbm, deg_hbm,
                 ib0, ib1, ib2, ib3, ib4, ib5, ib6, ib7, onesb, stage, hist,
                 is0, is1, is2, is3, is4, is5, is6, is7,
                 ss0, ss1, ss2, ss3, ss4, ss5, ss6, ss7):
    c = lax.axis_index("c")
    s = lax.axis_index("s")
    ibufs = (ib0, ib1, ib2, ib3, ib4, ib5, ib6, ib7)
    isems = (is0, is1, is2, is3, is4, is5, is6, is7)
    ssems = (ss0, ss1, ss2, ss3, ss4, ss5, ss6, ss7)

    for k in range(20):
        stage[pl.ds(k * 16, 16)] = _zero16()
    for k in range(8):
        onesb[pl.ds(k * 16, 16)] = _one16()
    win = c * _NHALF
    pltpu.sync_copy(stage, hist.at[pl.ds(win + s * 320, 320)])
    plsc.subcore_barrier()

    nj = ((_NCHUNK - 1 - s) // _NTILE) + 1

    def _doff(j):
        return (s + j * _NTILE) * _H

    @pl.loop(0, 11)
    def _steps(g):
        for p in range(8):
            j = g * 8 + p

            @pl.when((j >= 8) & (j - 8 < nj))
            def _():
                pltpu.make_async_copy(onesb, hist.at[ibufs[p]],
                                      ssems[p]).wait()

            @pl.when(j < nj)
            def _():
                pltpu.async_copy(dstf_hbm.at[pl.ds(_doff(j), _H)], ibufs[p],
                                 isems[p])

            q = (p - 4) % 8

            @pl.when((j >= 4) & (j - 4 < nj))
            def _():
                pltpu.make_async_copy(dstf_hbm.at[pl.ds(_doff(j - 4), _H)],
                                      ibufs[q], isems[q]).wait()
                pltpu.async_copy(onesb, hist.at[ibufs[q]], ssems[q],
                                 add=True)

    plsc.subcore_barrier()

    @pl.when(s < 15)
    def _():
        pltpu.sync_copy(hist.at[pl.ds(win + s * 320, 320)], stage)
        pltpu.sync_copy(stage, deg_hbm.at[pl.ds(win + s * 320, 320)])

    @pl.when(s == 15)
    def _():
        pltpu.sync_copy(hist.at[pl.ds(win + 4800, 200)],
                        stage.at[pl.ds(0, 200)])
        pltpu.sync_copy(stage.at[pl.ds(0, 200)],
                        deg_hbm.at[pl.ds(win + 4800, 200)])


_sc_deg = functools.partial(
    pl.kernel,
    out_type=jax.ShapeDtypeStruct((_N,), jnp.float32),
    mesh=_mesh,
    scratch_types=(
        [pltpu.VMEM((_H,), jnp.int32)] * 8
        + [pltpu.VMEM((_H,), jnp.float32)]
        + [pltpu.VMEM((320,), jnp.float32)]
        + [pltpu.VMEM_SHARED((10144,), jnp.float32)]
        + [pltpu.SemaphoreType.DMA] * 16
    ),
)(_sc_deg_body)


def _sc_scatter_body(hp_hbm, srcf_hbm, dstf_hbm, zeros_hbm, acc_hbm,
                     srcb0, srcb1, srcb2, srcb3,
                     dstb0, dstb1, dstb2, dstb3,
                     rows0, rows1, rows2, rows3, shared_acc,
                     isem0, isem1, isem2, isem3,
                     gsem0, gsem1, gsem2, gsem3,
                     ssem0, ssem1, ssem2, ssem3, zsem):
    c = lax.axis_index("c")
    s = lax.axis_index("s")
    row_off = c * _N

    srcbufs = (srcb0, srcb1, srcb2, srcb3)
    dstbufs = (dstb0, dstb1, dstb2, dstb3)
    rowbufs = (rows0, rows1, rows2, rows3)
    isems = (isem0, isem1, isem2, isem3)
    gsems = (gsem0, gsem1, gsem2, gsem3)
    ssems = (ssem0, ssem1, ssem2, ssem3)
    base = s * 640

    @pl.when(s < 15)
    def _():
        pltpu.async_copy(zeros_hbm, shared_acc.at[pl.ds(base, 640), :], zsem)

    @pl.when(s == 15)
    def _():
        pltpu.async_copy(zeros_hbm.at[pl.ds(0, 400), :],
                         shared_acc.at[pl.ds(base, 400), :], zsem)

    nj = _NCH2 // _NTILE

    def _soff(j):
        return (s + j * _NTILE) * _CH

    def _idx_load(j, p):
        pltpu.async_copy(srcf_hbm.at[pl.ds(_soff(j), _CH)], srcbufs[p],
                         isems[p])
        pltpu.async_copy(dstf_hbm.at[pl.ds(_soff(j), _CH)], dstbufs[p],
                         isems[p])

    def _idx_wait(j, p):
        pltpu.make_async_copy(srcf_hbm.at[pl.ds(_soff(j), _CH)], srcbufs[p],
                              isems[p]).wait()
        pltpu.make_async_copy(dstf_hbm.at[pl.ds(_soff(j), _CH)],
                              dstbufs[p], isems[p]).wait()

    def _gather_fire(p):
        for k in range(_CH // 16):
            srcbufs[p][pl.ds(k * 16, 16)] = (
                srcbufs[p][pl.ds(k * 16, 16)] + row_off)
        pltpu.async_copy(hp_hbm.at[srcbufs[p]], rowbufs[p], gsems[p])

    def _scatter_fire(p):
        pltpu.make_async_copy(hp_hbm.at[srcbufs[p]], rowbufs[p],
                              gsems[p]).wait()
        pltpu.async_copy(rowbufs[p], shared_acc.at[dstbufs[p]], ssems[p],
                         add=True)

    def _scatter_wait(p):
        pltpu.make_async_copy(rowbufs[p], shared_acc.at[dstbufs[p]],
                              ssems[p]).wait()

    _idx_load(0, 0)
    _idx_load(1, 1)
    _idx_wait(0, 0)
    _gather_fire(0)
    _idx_load(2, 2)
    _idx_wait(1, 1)
    _gather_fire(1)
    _idx_load(3, 3)
    _idx_wait(2, 2)
    _gather_fire(2)

    @pl.when(s < 15)
    def _():
        pltpu.make_async_copy(zeros_hbm,
                              shared_acc.at[pl.ds(base, 640), :],
                              zsem).wait()

    @pl.when(s == 15)
    def _():
        pltpu.make_async_copy(zeros_hbm.at[pl.ds(0, 400), :],
                              shared_acc.at[pl.ds(base, 400), :],
                              zsem).wait()

    plsc.subcore_barrier()
    _scatter_fire(0)

    @pl.loop(0, 32)
    def _steps(g):
        for pp in range(4):
            j = 4 + g * 4 + pp
            p = pp

            @pl.when(j - 4 < nj)
            def _():
                _scatter_wait(p)

            @pl.when(j < nj)
            def _():
                _idx_load(j, p)

            q = (p - 1) % 4

            @pl.when(j - 1 < nj)
            def _():
                _idx_wait(j - 1, q)
                _gather_fire(q)

            r = (p - 3) % 4

            @pl.when(j - 3 < nj)
            def _():
                _scatter_fire(r)

    plsc.subcore_barrier()

    @pl.when(s < 15)
    def _():
        pltpu.sync_copy(shared_acc.at[pl.ds(base, 640), :],
                        acc_hbm.at[pl.ds(row_off + base, 640), :])

    @pl.when(s == 15)
    def _():
        pltpu.sync_copy(shared_acc.at[pl.ds(base, 400), :],
                        acc_hbm.at[pl.ds(row_off + base, 400), :])


_sc_scatter = functools.partial(
    pl.kernel,
    out_type=jax.ShapeDtypeStruct((2 * _N, _H), jnp.float32),
    mesh=_mesh,
    scratch_types=(
        [pltpu.VMEM((_CH,), jnp.int32)] * 8
        + [pltpu.VMEM((_CH, _H), jnp.float32)] * 4
        + [pltpu.VMEM_SHARED((_N, _H), jnp.float32)]
        + [pltpu.SemaphoreType.DMA] * 13
    ),
)(_sc_scatter_body)


def _dis(deg_ref):
    return lax.rsqrt(deg_ref[...] + 1.0).reshape(_BM, 1)


def _tc1_body(x_ref, w_ref, deg_ref, hp_ref):
    dis = _dis(deg_ref)
    h = jnp.dot(x_ref[...], w_ref[...], preferred_element_type=jnp.float32)
    hp = h * dis
    hp_ref[0] = hp[:, :_H]
    hp_ref[1] = hp[:, _H:]


def _tc_mid_body(acc_ref, hp_ref, deg_ref, b_ref, w_ref, out_ref):
    dis = _dis(deg_ref)
    b = b_ref[...]
    hin_a = jnp.maximum(dis * (acc_ref[0] + hp_ref[0]) + b[:, :_H], 0.0)
    hin_b = jnp.maximum(dis * (acc_ref[1] + hp_ref[1]) + b[:, _H:], 0.0)
    hin = jnp.concatenate([hin_a, hin_b], axis=1)
    h = jnp.dot(hin, w_ref[...], preferred_element_type=jnp.float32)
    hp = h * dis
    out_ref[0] = hp[:, :_H]
    out_ref[1] = hp[:, _H:]


def _tc_final_body(acc_ref, hp_ref, deg_ref, b_ref, out_ref):
    dis = _dis(deg_ref)
    out_a = dis * (acc_ref[0] + hp_ref[0]) + b_ref[...][:, :_H]
    out_b = dis * (acc_ref[1] + hp_ref[1]) + b_ref[...][:, _H:]
    out_ref[...] = jnp.concatenate([out_a, out_b], axis=1)


_GRID = (_N + _BM - 1) // _BM

_spec_rows = pl.BlockSpec((_BM, _D), lambda i: (i, 0))
_spec_halves = pl.BlockSpec((2, _BM, _H), lambda i: (0, i, 0))
_spec_deg = pl.BlockSpec((_BM,), lambda i: (i,))
_spec_w = pl.BlockSpec((_D, _D), lambda i: (0, 0))
_spec_b = pl.BlockSpec((1, _D), lambda i: (0, 0))

_tc1 = pl.pallas_call(
    _tc1_body,
    grid=(_GRID,),
    in_specs=[_spec_rows, _spec_w, _spec_deg],
    out_specs=_spec_halves,
    out_shape=jax.ShapeDtypeStruct((2, _N, _H), jnp.float32),
)

_tc_mid = pl.pallas_call(
    _tc_mid_body,
    grid=(_GRID,),
    in_specs=[_spec_halves, _spec_halves, _spec_deg, _spec_b, _spec_w],
    out_specs=_spec_halves,
    out_shape=jax.ShapeDtypeStruct((2, _N, _H), jnp.float32),
)

_tc_final = pl.pallas_call(
    _tc_final_body,
    grid=(_GRID,),
    in_specs=[_spec_halves, _spec_halves, _spec_deg, _spec_b],
    out_specs=_spec_rows,
    out_shape=jax.ShapeDtypeStruct((_N, _D), jnp.float32),
)


def kernel(x, edge_index, W1, b1, W2, b2, W3, b3):
    srcf = edge_index[0]
    dstf = edge_index[1]

    degc = _sc_deg(dstf)

    b1r = b1.reshape(1, _D)
    b2r = b2.reshape(1, _D)
    b3r = b3.reshape(1, _D)

    hp1 = _tc1(x, W1, degc)
    zrows = jnp.zeros((640, _H), jnp.float32)
    acc1 = _sc_scatter(hp1.reshape(2 * _N, _H), srcf, dstf, zrows)
    hp2 = _tc_mid(acc1.reshape(2, _N, _H), hp1, degc, b1r, W2)
    acc2 = _sc_scatter(hp2.reshape(2 * _N, _H), srcf, dstf, zrows)
    hp3 = _tc_mid(acc2.reshape(2, _N, _H), hp2, degc, b2r, W3)
    acc3 = _sc_scatter(hp3.reshape(2 * _N, _H), srcf, dstf, zrows)
    out = _tc_final(acc3.reshape(2, _N, _H), hp3, degc, b3r)
    return out
